# baseline (speedup 1.0000x reference)
import jax
import jax.numpy as jnp
from jax import lax
from jax.experimental import pallas as pl
from jax.experimental.pallas import tpu as pltpu
from jax.experimental.pallas import tpu_sc as plsc

V = 4096
B, L = 1024, 20
N = B * L
NC, NS = 2, 16
NW = NC * NS
RPW = N // NW
W32 = RPW * 32
KC = 128
CSTEPS = W32 // KC
LANES = 16


def _sc_gather_body(table32_hbm, idx32_hbm, tgtc_hbm, tgtl_hbm,
                    out_hbm, tpart_hbm,
                    idx_v, tgtc_v, tgtl_v, acc_v,
                    rows0, rows1, sg0, sg1, so0, so1):
    wid = lax.axis_index("s") * NC + lax.axis_index("c")
    base = wid * W32
    bufs = (rows0, rows1)
    gsems = (sg0, sg1)
    osems = (so0, so1)

    pltpu.sync_copy(idx32_hbm.at[pl.ds(base, W32)], idx_v)
    pltpu.sync_copy(tgtc_hbm.at[pl.ds(wid * RPW, RPW)], tgtc_v)
    pltpu.sync_copy(tgtl_hbm.at[pl.ds(wid * RPW, RPW)], tgtl_v)

    def start_gather(g, b):
        src = table32_hbm.at[idx_v.at[pl.ds(g * KC, KC)]]
        pltpu.make_async_copy(src, bufs[b], gsems[b]).start()

    def wait_gather(b):
        pltpu.make_async_copy(
            table32_hbm.at[idx_v.at[pl.ds(0, KC)]], bufs[b], gsems[b]
        ).wait()

    def start_out(g, b):
        dst = out_hbm.at[pl.ds(base + g * KC, KC)]
        pltpu.make_async_copy(bufs[b], dst, osems[b]).start()

    def wait_out(b):
        pltpu.make_async_copy(
            bufs[b], out_hbm.at[pl.ds(base, KC)], osems[b]
        ).wait()

    start_gather(0, 0)

    def step(i, carry):
        for b in range(2):
            g = 2 * i + b
            ob = 1 - b

            @pl.when(g >= 1)
            def _():
                wait_out(ob)

            @pl.when(g + 1 < CSTEPS)
            def _():
                start_gather(g + 1, ob)

            wait_gather(b)
            start_out(g, b)
        return carry

    lax.fori_loop(0, CSTEPS // 2, step, 0)
    wait_out(1)

    lane = lax.iota(jnp.int32, LANES)
    acc = jnp.zeros((LANES,), jnp.float32)
    for j in range(RPW // KC):
        pltpu.make_async_copy(
            table32_hbm.at[tgtc_v.at[pl.ds(j * KC, KC)]], bufs[0], gsems[0]
        ).start()
        pltpu.make_async_copy(
            table32_hbm.at[tgtc_v.at[pl.ds(0, KC)]], bufs[0], gsems[0]
        ).wait()
        for j2 in range(KC // LANES):
            rows = lane + j2 * LANES
            cols = tgtl_v[pl.ds(j * KC + j2 * LANES, LANES)]
            acc = acc + plsc.load_gather(bufs[0], [rows, cols])
    acc_v[...] = acc
    pltpu.sync_copy(acc_v, tpart_hbm.at[pl.ds(wid * LANES, LANES)])


_sc_gather = pl.kernel(
    _sc_gather_body,
    out_type=(
        jax.ShapeDtypeStruct((N * 32, 128), jnp.float32),
        jax.ShapeDtypeStruct((NW * LANES,), jnp.float32),
    ),
    mesh=plsc.VectorSubcoreMesh(core_axis_name="c", subcore_axis_name="s"),
    compiler_params=pltpu.CompilerParams(needs_layout_passes=False),
    scratch_types=[
        pltpu.VMEM((W32,), jnp.int32),
        pltpu.VMEM((RPW,), jnp.int32),
        pltpu.VMEM((RPW,), jnp.int32),
        pltpu.VMEM((LANES,), jnp.float32),
        pltpu.VMEM((KC, 128), jnp.float32),
        pltpu.VMEM((KC, 128), jnp.float32),
        pltpu.SemaphoreType.DMA,
        pltpu.SemaphoreType.DMA,
        pltpu.SemaphoreType.DMA,
        pltpu.SemaphoreType.DMA,
    ],
)


def _sc_lse_gather_body(lse_hbm, idx_hbm, part_hbm, lse_v, idx_v, acc_v):
    wid = lax.axis_index("s") * NC + lax.axis_index("c")
    base = wid * RPW
    pltpu.sync_copy(lse_hbm, lse_v)
    pltpu.sync_copy(idx_hbm.at[pl.ds(base, RPW)], idx_v)

    acc = jnp.zeros((LANES,), jnp.float32)
    for j in range(RPW // LANES):
        iv = idx_v[pl.ds(j * LANES, LANES)]
        acc = acc + plsc.load_gather(lse_v, [iv])
    acc_v[...] = acc
    pltpu.sync_copy(acc_v, part_hbm.at[pl.ds(wid * LANES, LANES)])


_sc_lse_gather = pl.kernel(
    _sc_lse_gather_body,
    out_type=jax.ShapeDtypeStruct((NW * LANES,), jnp.float32),
    mesh=plsc.VectorSubcoreMesh(core_axis_name="c", subcore_axis_name="s"),
    compiler_params=pltpu.CompilerParams(needs_layout_passes=False),
    scratch_types=[
        pltpu.VMEM((V,), jnp.float32),
        pltpu.VMEM((RPW,), jnp.int32),
        pltpu.VMEM((LANES,), jnp.float32),
    ],
)


LSE_BLK = 256
LSE_BLKS = V // LSE_BLK


def _tc_lse_body(table_ref, lse_ref):
    x = table_ref[...]
    m = jnp.max(x, axis=1)
    s = jnp.sum(jnp.exp(x - m[:, None]), axis=1)
    lse_ref[...] = jnp.log(s) + m


_tc_lse = pl.pallas_call(
    _tc_lse_body,
    grid=(LSE_BLKS,),
    in_specs=[pl.BlockSpec((LSE_BLK, V), lambda i: (i, 0))],
    out_specs=pl.BlockSpec((LSE_BLK,), lambda i: (i,)),
    out_shape=jax.ShapeDtypeStruct((V,), jnp.float32),
)


def _tc_combine_body(lsep_ref, tgtp_ref, out_ref):
    out_ref[0, 0] = (jnp.sum(lsep_ref[...]) - jnp.sum(tgtp_ref[...])) / N


_tc_combine = pl.pallas_call(
    _tc_combine_body,
    out_specs=pl.BlockSpec(memory_space=pltpu.SMEM),
    out_shape=jax.ShapeDtypeStruct((1, 1), jnp.float32),
)


def kernel(indices, targets, table):
    idx = indices.reshape(-1).astype(jnp.int32)
    tgt = targets.reshape(-1).astype(jnp.int32)
    table32 = table.reshape(V * 32, 128)
    idx32 = (idx[:, None] * 32 + jnp.arange(32, dtype=jnp.int32)).reshape(-1)
    tgtc = idx * 32 + (tgt >> 7)
    tgtl = tgt & 127
    out32, tgt_parts = _sc_gather(table32, idx32, tgtc, tgtl)
    lse = _tc_lse(table)
    lse_parts = _sc_lse_gather(lse, idx)
    loss = _tc_combine(lse_parts, tgt_parts)
    return out32.reshape(indices.shape + (V,)), loss[0, 0]
